# baseline (speedup 1.0000x reference)
import functools
import jax
import jax.numpy as jnp
from jax import lax
from jax.experimental import pallas as pl
from jax.experimental.pallas import tpu as pltpu
from jax.experimental.pallas import tpu_sc as plsc

N = 10000
E = 320000
D = 128
C1 = 128
C2 = 64

NS = 16
LANES = 16
N_PAD = 10240
ROWS_PER_TILE = N_PAD // NS
E_TILE = E // NS
NBUF = 4
LOOK = 3
BI = 8
CH = 128
EC = 125
SEGC = 16
NSEG = 10
NIB = 3
HW = 64

ROWB = 1024


def _make_sc_conv(n_pass, pool=False):
  mesh = plsc.VectorSubcoreMesh(core_axis_name="c", subcore_axis_name="s")
  P = n_pass
  assert not pool or P == 1
  n_bias = 1 if pool else 0
  out_shape = (NS, HW) if pool else (N_PAD, HW)
  LAST = N - (NS - 1) * ROWS_PER_TILE

  @functools.partial(
      pl.kernel,
      out_type=[jax.ShapeDtypeStruct(out_shape, jnp.float32)] * (2 * P),
      mesh=mesh,
      compiler_params=pltpu.CompilerParams(use_tc_tiling_on_sc=False),
      scratch_types=[
          pltpu.VMEM_SHARED((N_PAD, HW), jnp.float32),
          pltpu.VMEM_SHARED((N_PAD, HW), jnp.float32),
          [pltpu.VMEM((SEGC, EC), jnp.int32)] * NIB,
          [pltpu.VMEM((SEGC, EC), jnp.int32)] * NIB,
          [pltpu.VMEM((CH, HW), jnp.float32)] * NBUF,
          [pltpu.SemaphoreType.DMA] * NBUF,
          [pltpu.SemaphoreType.DMA] * NBUF,
          pltpu.SemaphoreType.DMA,
          pltpu.VMEM((HW,), jnp.float32),
      ],
  )
  def sc_conv(*refs):
    t1, t2 = refs[0:2]
    s1_hbm, d1_hbm, s2_hbm, d2_hbm = refs[2:6]
    b_hbm = refs[6:6 + n_bias]
    o1 = refs[6 + n_bias:6 + n_bias + P]
    o2 = refs[6 + n_bias + P:6 + n_bias + 2 * P]
    (table, acc, src_v, dst_v, bufs, gsem, ssem, isem,
     bbuf) = refs[6 + n_bias + 2 * P:]
    c = lax.axis_index("c")
    s = lax.axis_index("s")
    row0 = s * ROWS_PER_TILE

    def gather(k, b, gb):
      pltpu.async_copy(table.at[src_v[gb].at[k]], bufs[b].at[pl.ds(0, EC)],
                       gsem[b])

    def wait_g(b):
      pltpu.make_async_copy(table.at[src_v[0].at[0]], bufs[b].at[pl.ds(0, EC)],
                            gsem[b]).wait()

    def scat(k, b, gb):
      pltpu.async_copy(bufs[b].at[pl.ds(0, EC)], acc.at[dst_v[gb].at[k]],
                       ssem[b], add=True)

    def wait_s(b):
      pltpu.make_async_copy(bufs[b].at[pl.ds(0, EC)], acc.at[dst_v[0].at[0]],
                            ssem[b]).wait()

    def load_idx(g, gb):
      @pl.when(c == 0)
      def _():
        pltpu.async_copy(s1_hbm.at[s, g], src_v[gb], isem)
        pltpu.async_copy(d1_hbm.at[s, g], dst_v[gb], isem)
      @pl.when(c != 0)
      def _():
        pltpu.async_copy(s2_hbm.at[s, g], src_v[gb], isem)
        pltpu.async_copy(d2_hbm.at[s, g], dst_v[gb], isem)

    def wait_idx(gb):
      pltpu.make_async_copy(s1_hbm.at[s, 0], src_v[gb], isem).wait()
      pltpu.make_async_copy(s1_hbm.at[s, 0], dst_v[gb], isem).wait()

    zero16 = jnp.zeros((LANES,), jnp.float32)
    nhops = ROWS_PER_TILE // CH
    sl640 = pl.ds(row0, ROWS_PER_TILE)

    if pool:
      pltpu.sync_copy(b_hbm[0].at[c], bbuf)

    for p in range(P):
      @pl.loop(0, CH)
      def _(i):
        for j in range(HW // LANES):
          bufs[0][i, pl.ds(j * LANES, LANES)] = zero16

      if P > 1:
        col = pl.ds(p * HW, HW)
        def stage_src(t_ref, rows):
          return t_ref.at[pl.ds(row0, rows), col]
      else:
        def stage_src(t_ref, rows):
          return t_ref.at[pl.ds(row0, rows)]

      def stage(fn):
        @pl.when(jnp.logical_and(c == 0, s != NS - 1))
        def _():
          fn(stage_src(t1, ROWS_PER_TILE), table.at[sl640])
        @pl.when(jnp.logical_and(c != 0, s != NS - 1))
        def _():
          fn(stage_src(t2, ROWS_PER_TILE), table.at[sl640])
        @pl.when(jnp.logical_and(c == 0, s == NS - 1))
        def _():
          fn(stage_src(t1, LAST), table.at[pl.ds(row0, LAST)])
        @pl.when(jnp.logical_and(c != 0, s == NS - 1))
        def _():
          fn(stage_src(t2, LAST), table.at[pl.ds(row0, LAST)])

      stage(lambda a, b: pltpu.async_copy(a, b, gsem[0]))
      for k in range(nhops):
        pltpu.async_copy(bufs[0], acc.at[pl.ds(row0 + k * CH, CH)], ssem[0])
      stage(lambda a, b: pltpu.make_async_copy(a, b, gsem[0]).wait())
      for k in range(nhops):
        pltpu.make_async_copy(bufs[0], acc.at[pl.ds(row0, CH)],
                              ssem[0]).wait()

      plsc.subcore_barrier()

      load_idx(0, 0)
      wait_idx(0)
      if NSEG > 1:
        load_idx(1, 1)
      for q in range(LOOK):
        gather(q, q, 0)
      for g in range(NSEG):
        gb = g % NIB
        @pl.loop(0, SEGC // BI)
        def _(o, g=g, gb=gb):
          for k2 in range(BI):
            k = o * BI + k2
            b = k2 % NBUF
            wait_g(b)
            scat(k, b, gb)
            nb = (k2 + LOOK) % NBUF
            @pl.when(k + LOOK < SEGC)
            def _():
              if g == 0:
                @pl.when(k >= NBUF - LOOK)
                def _():
                  wait_s(nb)
                gather(k + LOOK, nb, gb)
              else:
                wait_s(nb)
                gather(k + LOOK, nb, gb)
        if g + 1 < NSEG:
          ngb = (g + 1) % NIB
          wait_idx(ngb)
          if g + 2 < NSEG:
            load_idx(g + 2, (g + 2) % NIB)
          for b in range(LOOK):
            wait_s(b)
          for q in range(LOOK):
            gather(q, q, ngb)
        else:
          for b in range(NBUF):
            wait_s(b)

      plsc.subcore_barrier()

      if pool:
        carry = tuple(jnp.zeros((LANES,), jnp.float32)
                      for _ in range(HW // LANES))
        for h in range(nhops):
          pltpu.sync_copy(acc.at[pl.ds(row0 + h * CH, CH)], bufs[0])
          def rowloop(r, cr, h=h):
            grow = row0 + h * CH + r
            out = []
            for j in range(HW // LANES):
              v = (bufs[0][r, pl.ds(j * LANES, LANES)]
                   + bbuf[pl.ds(j * LANES, LANES)])
              v = jnp.where(grow < N, jnp.maximum(v, 0.0), 0.0)
              out.append(cr[j] + v)
            return tuple(out)
          carry = lax.fori_loop(0, CH, rowloop, carry)
        for j in range(HW // LANES):
          bbuf[pl.ds(j * LANES, LANES)] = carry[j]
        @pl.when(c == 0)
        def _():
          pltpu.sync_copy(bbuf, o1[0].at[s])
        @pl.when(c != 0)
        def _():
          pltpu.sync_copy(bbuf, o2[0].at[s])
      else:
        @pl.when(c == 0)
        def _():
          pltpu.sync_copy(acc.at[sl640], o1[p].at[sl640])
        @pl.when(c != 0)
        def _():
          pltpu.sync_copy(acc.at[sl640], o2[p].at[sl640])
      if p + 1 < P:
        plsc.subcore_barrier()

  return sc_conv


_make_sc_conv = functools.lru_cache(maxsize=None)(_make_sc_conv)


def _mid_body(a1a_ref, a1b_ref, a2a_ref, a2b_ref, wa1_ref, wa2_ref,
              b1_ref, b2_ref, w1_ref, w2_ref, z1_ref, z2_ref):
  a1 = jnp.concatenate([a1a_ref[...], a1b_ref[...]], axis=1)
  a2 = jnp.concatenate([a2a_ref[...], a2b_ref[...]], axis=1)
  h1 = jnp.maximum(jnp.dot(a1, wa1_ref[...],
                           preferred_element_type=jnp.float32)
                   + b1_ref[...], 0.0)
  h2 = jnp.maximum(jnp.dot(a2, wa2_ref[...],
                           preferred_element_type=jnp.float32)
                   + b2_ref[...], 0.0)
  z1_ref[...] = jnp.dot(h1, w1_ref[...], preferred_element_type=jnp.float32)
  z2_ref[...] = jnp.dot(h2, w2_ref[...], preferred_element_type=jnp.float32)


def _mid(a1a, a1b, a2a, a2b, wa1, wa2, b1, b2, w1, w2):
  return pl.pallas_call(
      _mid_body,
      grid=(N_PAD // ROWB,),
      in_specs=[
          pl.BlockSpec((ROWB, HW), lambda i: (i, 0)),
          pl.BlockSpec((ROWB, HW), lambda i: (i, 0)),
          pl.BlockSpec((ROWB, HW), lambda i: (i, 0)),
          pl.BlockSpec((ROWB, HW), lambda i: (i, 0)),
          pl.BlockSpec((D, C1), lambda i: (0, 0)),
          pl.BlockSpec((D, C1), lambda i: (0, 0)),
          pl.BlockSpec((1, C1), lambda i: (0, 0)),
          pl.BlockSpec((1, C1), lambda i: (0, 0)),
          pl.BlockSpec((C1, C2), lambda i: (0, 0)),
          pl.BlockSpec((C1, C2), lambda i: (0, 0)),
      ],
      out_specs=[
          pl.BlockSpec((ROWB, C2), lambda i: (i, 0)),
          pl.BlockSpec((ROWB, C2), lambda i: (i, 0)),
      ],
      out_shape=[jax.ShapeDtypeStruct((N_PAD, C2), jnp.float32),
                 jax.ShapeDtypeStruct((N_PAD, C2), jnp.float32)],
  )(a1a, a1b, a2a, a2b, wa1, wa2, b1, b2, w1, w2)


def _head_body(p1_ref, p2_ref, wd1_ref, bd1_ref,
               wd2_ref, bd2_ref, wo_ref, bo_ref, o_ref):
  g1 = jnp.sum(p1_ref[...], axis=0, keepdims=True) * (1.0 / N)
  g2 = jnp.sum(p2_ref[...], axis=0, keepdims=True) * (1.0 / N)
  g = jnp.concatenate([g1, g2], axis=1)
  t = jnp.maximum(jnp.dot(g, wd1_ref[...],
                          preferred_element_type=jnp.float32)
                  + bd1_ref[...], 0.0)
  t = jnp.maximum(jnp.dot(t, wd2_ref[...],
                          preferred_element_type=jnp.float32)
                  + bd2_ref[...], 0.0)
  u = jnp.dot(t, wo_ref[...], preferred_element_type=jnp.float32) + bo_ref[...]
  o_ref[...] = 1.0 / (1.0 + jnp.exp(-u))


def _head(p1, p2, wd1, bd1, wd2, bd2, wo, bo):
  return pl.pallas_call(
      _head_body,
      out_shape=jax.ShapeDtypeStruct((1, 1), jnp.float32),
  )(p1, p2, wd1, bd1, wd2, bd2, wo, bo)


def _shape_edges(e):
  return (e[0].reshape(NS, NSEG, SEGC, EC),
          e[1].reshape(NS, NSEG, SEGC, EC))


@jax.jit
def kernel(x1, edge_index1, x2, edge_index2,
           W1a, b1a, W1b, b1b, W2a, b2a, W2b, b2b,
           Wd1, bd1, Wd2, bd2, Wout, bout):
  src1, dst1 = _shape_edges(edge_index1)
  src2, dst2 = _shape_edges(edge_index2)

  a1a, a1b, a2a, a2b = _make_sc_conv(2)(x1, x2, src1, dst1, src2, dst2)
  z1, z2 = _mid(a1a, a1b, a2a, a2b, W1a, W2a,
                b1a.reshape(1, C1), b2a.reshape(1, C1), W1b, W2b)
  p1, p2 = _make_sc_conv(1, True)(z1, z2, src1, dst1, src2, dst2,
                                  jnp.stack([b1b, b2b]))
  out = _head(p1, p2,
              Wd1, bd1.reshape(1, C2), Wd2, bd2.reshape(1, C2),
              Wout, bout.reshape(1, 1))
  return out.reshape(1)


# speedup vs baseline: 1.1396x; 1.1396x over previous
import functools
import jax
import jax.numpy as jnp
from jax import lax
from jax.experimental import pallas as pl
from jax.experimental.pallas import tpu as pltpu
from jax.experimental.pallas import tpu_sc as plsc

N = 10000
E = 320000
D = 128
C1 = 128
C2 = 64

NS = 16
LANES = 16
N_PAD = 10240
ROWS_PER_TILE = N_PAD // NS
E_TILE = E // NS
NBUF = 4
LOOK = 3
BI = 8
CH = 128
EC = 125
SEGC = 40
NSEG = 4
HW = 64

ROWB = 1024


def _make_sc_conv(n_pass, pool=False):
  mesh = plsc.VectorSubcoreMesh(core_axis_name="c", subcore_axis_name="s")
  P = n_pass
  assert not pool or P == 1
  n_bias = 1 if pool else 0
  out_shape = (NS, HW) if pool else (N_PAD, HW)
  LAST = N - (NS - 1) * ROWS_PER_TILE

  @functools.partial(
      pl.kernel,
      out_type=[jax.ShapeDtypeStruct(out_shape, jnp.float32)] * (2 * P),
      mesh=mesh,
      compiler_params=pltpu.CompilerParams(use_tc_tiling_on_sc=False),
      scratch_types=[
          pltpu.VMEM_SHARED((N_PAD, HW), jnp.float32),
          pltpu.VMEM_SHARED((N_PAD, HW), jnp.float32),
          pltpu.VMEM((SEGC, EC), jnp.int32),
          pltpu.VMEM((SEGC, EC), jnp.int32),
          [pltpu.VMEM((CH, HW), jnp.float32)] * NBUF,
          [pltpu.SemaphoreType.DMA] * NBUF,
          [pltpu.SemaphoreType.DMA] * NBUF,
          pltpu.VMEM((HW,), jnp.float32),
      ],
  )
  def sc_conv(*refs):
    t1, t2 = refs[0:2]
    s1_hbm, d1_hbm, s2_hbm, d2_hbm = refs[2:6]
    b_hbm = refs[6:6 + n_bias]
    o1 = refs[6 + n_bias:6 + n_bias + P]
    o2 = refs[6 + n_bias + P:6 + n_bias + 2 * P]
    table, acc, src_v, dst_v, bufs, gsem, ssem, bbuf = refs[6 + n_bias + 2 * P:]
    c = lax.axis_index("c")
    s = lax.axis_index("s")
    row0 = s * ROWS_PER_TILE

    def gather(k, b):
      pltpu.async_copy(table.at[src_v.at[k]], bufs[b].at[pl.ds(0, EC)],
                       gsem[b])

    def wait_g(b):
      pltpu.make_async_copy(table.at[src_v.at[0]], bufs[b].at[pl.ds(0, EC)],
                            gsem[b]).wait()

    def scat(k, b):
      pltpu.async_copy(bufs[b].at[pl.ds(0, EC)], acc.at[dst_v.at[k]],
                       ssem[b], add=True)

    def wait_s(b):
      pltpu.make_async_copy(bufs[b].at[pl.ds(0, EC)], acc.at[dst_v.at[0]],
                            ssem[b]).wait()

    zero16 = jnp.zeros((LANES,), jnp.float32)
    nhops = ROWS_PER_TILE // CH
    sl640 = pl.ds(row0, ROWS_PER_TILE)

    if pool:
      pltpu.sync_copy(b_hbm[0].at[c], bbuf)

    for p in range(P):
      @pl.loop(0, CH)
      def _(i):
        for j in range(HW // LANES):
          bufs[0][i, pl.ds(j * LANES, LANES)] = zero16

      if P > 1:
        col = pl.ds(p * HW, HW)
        def stage_src(t_ref, rows):
          return t_ref.at[pl.ds(row0, rows), col]
      else:
        def stage_src(t_ref, rows):
          return t_ref.at[pl.ds(row0, rows)]

      def stage(fn):
        @pl.when(jnp.logical_and(c == 0, s != NS - 1))
        def _():
          fn(stage_src(t1, ROWS_PER_TILE), table.at[sl640])
        @pl.when(jnp.logical_and(c != 0, s != NS - 1))
        def _():
          fn(stage_src(t2, ROWS_PER_TILE), table.at[sl640])
        @pl.when(jnp.logical_and(c == 0, s == NS - 1))
        def _():
          fn(stage_src(t1, LAST), table.at[pl.ds(row0, LAST)])
        @pl.when(jnp.logical_and(c != 0, s == NS - 1))
        def _():
          fn(stage_src(t2, LAST), table.at[pl.ds(row0, LAST)])

      stage(lambda a, b: pltpu.async_copy(a, b, gsem[0]))
      for k in range(nhops):
        pltpu.async_copy(bufs[0], acc.at[pl.ds(row0 + k * CH, CH)], ssem[0])
      stage(lambda a, b: pltpu.make_async_copy(a, b, gsem[0]).wait())
      for k in range(nhops):
        pltpu.make_async_copy(bufs[0], acc.at[pl.ds(row0, CH)],
                              ssem[0]).wait()

      plsc.subcore_barrier()

      @pl.loop(0, NSEG)
      def _(g):
        @pl.when(c == 0)
        def _():
          pltpu.sync_copy(s1_hbm.at[s, g], src_v)
          pltpu.sync_copy(d1_hbm.at[s, g], dst_v)
        @pl.when(c != 0)
        def _():
          pltpu.sync_copy(s2_hbm.at[s, g], src_v)
          pltpu.sync_copy(d2_hbm.at[s, g], dst_v)
        for q in range(LOOK):
          gather(q, q)
        @pl.loop(0, SEGC // BI)
        def _(o):
          for k2 in range(BI):
            k = o * BI + k2
            b = k2 % NBUF
            wait_g(b)
            scat(k, b)
            nb = (k2 + LOOK) % NBUF
            @pl.when(k + LOOK < SEGC)
            def _():
              @pl.when(k >= NBUF - LOOK)
              def _():
                wait_s(nb)
              gather(k + LOOK, nb)
        for b in range(NBUF):
          wait_s(b)

      plsc.subcore_barrier()

      if pool:
        carry = tuple(jnp.zeros((LANES,), jnp.float32)
                      for _ in range(HW // LANES))
        for h in range(nhops):
          pltpu.sync_copy(acc.at[pl.ds(row0 + h * CH, CH)], bufs[0])
          def rowloop(r, cr, h=h):
            grow = row0 + h * CH + r
            out = []
            for j in range(HW // LANES):
              v = (bufs[0][r, pl.ds(j * LANES, LANES)]
                   + bbuf[pl.ds(j * LANES, LANES)])
              v = jnp.where(grow < N, jnp.maximum(v, 0.0), 0.0)
              out.append(cr[j] + v)
            return tuple(out)
          carry = lax.fori_loop(0, CH, rowloop, carry)
        for j in range(HW // LANES):
          bbuf[pl.ds(j * LANES, LANES)] = carry[j]
        @pl.when(c == 0)
        def _():
          pltpu.sync_copy(bbuf, o1[0].at[s])
        @pl.when(c != 0)
        def _():
          pltpu.sync_copy(bbuf, o2[0].at[s])
      else:
        @pl.when(c == 0)
        def _():
          pltpu.sync_copy(acc.at[sl640], o1[p].at[sl640])
        @pl.when(c != 0)
        def _():
          pltpu.sync_copy(acc.at[sl640], o2[p].at[sl640])
      if p + 1 < P:
        plsc.subcore_barrier()

  return sc_conv


_make_sc_conv = functools.lru_cache(maxsize=None)(_make_sc_conv)


def _mid_body(a1a_ref, a1b_ref, a2a_ref, a2b_ref, wa1_ref, wa2_ref,
              b1_ref, b2_ref, w1_ref, w2_ref, z1_ref, z2_ref):
  a1 = jnp.concatenate([a1a_ref[...], a1b_ref[...]], axis=1)
  a2 = jnp.concatenate([a2a_ref[...], a2b_ref[...]], axis=1)
  h1 = jnp.maximum(jnp.dot(a1, wa1_ref[...],
                           preferred_element_type=jnp.float32)
                   + b1_ref[...], 0.0)
  h2 = jnp.maximum(jnp.dot(a2, wa2_ref[...],
                           preferred_element_type=jnp.float32)
                   + b2_ref[...], 0.0)
  z1_ref[...] = jnp.dot(h1, w1_ref[...], preferred_element_type=jnp.float32)
  z2_ref[...] = jnp.dot(h2, w2_ref[...], preferred_element_type=jnp.float32)


def _mid(a1a, a1b, a2a, a2b, wa1, wa2, b1, b2, w1, w2):
  return pl.pallas_call(
      _mid_body,
      grid=(N_PAD // ROWB,),
      in_specs=[
          pl.BlockSpec((ROWB, HW), lambda i: (i, 0)),
          pl.BlockSpec((ROWB, HW), lambda i: (i, 0)),
          pl.BlockSpec((ROWB, HW), lambda i: (i, 0)),
          pl.BlockSpec((ROWB, HW), lambda i: (i, 0)),
          pl.BlockSpec((D, C1), lambda i: (0, 0)),
          pl.BlockSpec((D, C1), lambda i: (0, 0)),
          pl.BlockSpec((1, C1), lambda i: (0, 0)),
          pl.BlockSpec((1, C1), lambda i: (0, 0)),
          pl.BlockSpec((C1, C2), lambda i: (0, 0)),
          pl.BlockSpec((C1, C2), lambda i: (0, 0)),
      ],
      out_specs=[
          pl.BlockSpec((ROWB, C2), lambda i: (i, 0)),
          pl.BlockSpec((ROWB, C2), lambda i: (i, 0)),
      ],
      out_shape=[jax.ShapeDtypeStruct((N_PAD, C2), jnp.float32),
                 jax.ShapeDtypeStruct((N_PAD, C2), jnp.float32)],
  )(a1a, a1b, a2a, a2b, wa1, wa2, b1, b2, w1, w2)


def _head_body(p1_ref, p2_ref, wd1_ref, bd1_ref,
               wd2_ref, bd2_ref, wo_ref, bo_ref, o_ref):
  g1 = jnp.sum(p1_ref[...], axis=0, keepdims=True) * (1.0 / N)
  g2 = jnp.sum(p2_ref[...], axis=0, keepdims=True) * (1.0 / N)
  g = jnp.concatenate([g1, g2], axis=1)
  t = jnp.maximum(jnp.dot(g, wd1_ref[...],
                          preferred_element_type=jnp.float32)
                  + bd1_ref[...], 0.0)
  t = jnp.maximum(jnp.dot(t, wd2_ref[...],
                          preferred_element_type=jnp.float32)
                  + bd2_ref[...], 0.0)
  u = jnp.dot(t, wo_ref[...], preferred_element_type=jnp.float32) + bo_ref[...]
  o_ref[...] = 1.0 / (1.0 + jnp.exp(-u))


def _head(p1, p2, wd1, bd1, wd2, bd2, wo, bo):
  return pl.pallas_call(
      _head_body,
      out_shape=jax.ShapeDtypeStruct((1, 1), jnp.float32),
  )(p1, p2, wd1, bd1, wd2, bd2, wo, bo)


def _shape_edges(e):
  return (e[0].reshape(NS, NSEG, SEGC, EC),
          e[1].reshape(NS, NSEG, SEGC, EC))


@jax.jit
def kernel(x1, edge_index1, x2, edge_index2,
           W1a, b1a, W1b, b1b, W2a, b2a, W2b, b2b,
           Wd1, bd1, Wd2, bd2, Wout, bout):
  src1, dst1 = _shape_edges(edge_index1)
  src2, dst2 = _shape_edges(edge_index2)

  a1a, a1b, a2a, a2b = _make_sc_conv(2)(x1, x2, src1, dst1, src2, dst2)
  z1, z2 = _mid(a1a, a1b, a2a, a2b, W1a, W2a,
                b1a.reshape(1, C1), b2a.reshape(1, C1), W1b, W2b)
  p1, p2 = _make_sc_conv(1, True)(z1, z2, src1, dst1, src2, dst2,
                                  jnp.stack([b1b, b2b]))
  out = _head(p1, p2,
              Wd1, bd1.reshape(1, C2), Wd2, bd2.reshape(1, C2),
              Wout, bout.reshape(1, 1))
  return out.reshape(1)
